# baseline (speedup 1.0000x reference)
import jax
import jax.numpy as jnp
from jax.experimental import pallas as pl

F32 = jnp.float32
BF16 = jnp.bfloat16
BN_EPS = 1e-5


def _split(v):
    hi = v.astype(BF16)
    lo = (v - hi.astype(F32)).astype(BF16)
    return hi, lo


def _dotg(a, b, tb=False):
    dims = (((1,), (1 if tb else 0,)), ((), ()))
    return jax.lax.dot_general(a, b, dims, preferred_element_type=F32)


def _bf(v):
    return v if v.dtype == BF16 else v.astype(BF16)


def _mm(a, b, *, trans_b=False, add=None, bias=None, gamma=None, beta=None,
        epi="none", bk=None, bn=None, out_dtype=F32, qk_scale=None):
    M, K = a.shape
    N = b.shape[0] if trans_b else b.shape[1]
    bk = bk or K
    bn = bn or N
    gk, gn = K // bk, N // bn
    assert out_dtype == F32 or gk == 1

    a_spec = pl.BlockSpec((M, bk), lambda n, k: (0, k))
    if trans_b:
        b_spec = pl.BlockSpec((bn, bk), lambda n, k: (n, k))
    else:
        b_spec = pl.BlockSpec((bk, bn), lambda n, k: (k, n))
    vec_spec = pl.BlockSpec((1, bn), lambda n, k: (0, n))

    in_specs = [a_spec]
    args = [a]
    if add is not None:
        in_specs.append(a_spec)
        args.append(add)
    in_specs.append(b_spec)
    args.append(b)
    for vv in (bias, gamma, beta):
        if vv is not None:
            in_specs.append(vec_spec)
            args.append(vv.reshape(1, N))

    has_add = add is not None
    has_bias = bias is not None
    has_bn = gamma is not None

    def body(*refs):
        refs = list(refs)
        a_ref = refs.pop(0)
        add_ref = refs.pop(0) if has_add else None
        b_ref = refs.pop(0)
        bias_ref = refs.pop(0) if has_bias else None
        g_ref = refs.pop(0) if has_bn else None
        be_ref = refs.pop(0) if has_bn else None
        o_ref = refs.pop(0)

        def epilogue(r):
            if has_bias:
                r = r + bias_ref[...]
            if has_bn:
                mcol = jnp.mean(r, axis=0, keepdims=True)
                vcol = jnp.mean((r - mcol) ** 2, axis=0, keepdims=True)
                r = (r - mcol) / jnp.sqrt(vcol + BN_EPS) * g_ref[...] \
                    + be_ref[...]
            if qk_scale is not None:
                s, nq = qk_scale
                r = jnp.where(pl.program_id(0) < nq, r / s, r)
            if epi == "relu":
                r = jnp.maximum(r, 0.0)
            elif epi == "sigmoid":
                r = jax.nn.sigmoid(r)
            elif epi == "expclip":
                r = jnp.clip(jnp.exp(r), 1e-5, 1e6)
            elif epi == "softplusclip":
                r = jnp.clip(jax.nn.softplus(r), 1e-4, 1e4)
            return r

        av = a_ref[...]
        if has_add:
            av = av + add_ref[...]
        contrib = _dotg(_bf(av), _bf(b_ref[...]), trans_b)

        if gk == 1:
            o_ref[...] = epilogue(contrib).astype(out_dtype)
        else:
            k = pl.program_id(1)

            @pl.when(k == 0)
            def _zero():
                o_ref[...] = jnp.zeros_like(o_ref)

            o_ref[...] += contrib

            @pl.when(k == gk - 1)
            def _epi():
                o_ref[...] = epilogue(o_ref[...])

    return pl.pallas_call(
        body,
        grid=(gn, gk),
        in_specs=in_specs,
        out_specs=pl.BlockSpec((M, bn), lambda n, k: (0, n)),
        out_shape=jax.ShapeDtypeStruct((M, N), out_dtype),
    )(*args)


def _gnn_fused(adj_bf, inp_bf, w_bf, epi="relu", bk=512, with_softmax=False):
    M = adj_bf.shape[0]
    kin = inp_bf.shape[1]
    N = w_bf.shape[1]
    gk = M // bk

    in_specs = [pl.BlockSpec((M, bk), lambda k: (0, k)),
                pl.BlockSpec((bk, kin), lambda k: (k, 0)),
                pl.BlockSpec((kin, N), lambda k: (0, 0))]
    args = [adj_bf, inp_bf, w_bf]

    if with_softmax:
        out_specs = [pl.BlockSpec((M, N), lambda k: (0, 0))] * 2
        out_shape = [jax.ShapeDtypeStruct((M, N), F32)] * 2
    else:
        out_specs = pl.BlockSpec((M, N), lambda k: (0, 0))
        out_shape = jax.ShapeDtypeStruct((M, N), F32)

    def body(adj_ref, inp_ref, w_ref, o_ref, *rest):
        k = pl.program_id(0)

        @pl.when(k == 0)
        def _zero():
            o_ref[...] = jnp.zeros_like(o_ref)

        t_blk = _dotg(_bf(inp_ref[...]), w_ref[...])
        o_ref[...] += _dotg(adj_ref[...], t_blk.astype(BF16))

        @pl.when(k == gk - 1)
        def _epi():
            r = o_ref[...]
            if epi == "relu":
                r = jnp.maximum(r, 0.0)
            o_ref[...] = r
            if with_softmax:
                mx = jnp.max(r, axis=1, keepdims=True)
                pr = jnp.exp(r - mx)
                rest[0][...] = pr / jnp.sum(pr, axis=1, keepdims=True)

    return pl.pallas_call(
        body,
        grid=(gk,),
        in_specs=in_specs,
        out_specs=out_specs,
        out_shape=out_shape,
    )(*args)


def _attention(qkv, heads, ep, t, bq=256):
    def body(q_ref, k_ref, v_ref, o_ref):
        kv = k_ref[...]
        vv = v_ref[...]
        for i in range(t // bq):
            sl = pl.ds(i * bq, bq)
            s = _dotg(q_ref[sl, :], kv, True)
            m = jnp.max(s, axis=1, keepdims=True)
            pr = jnp.exp(s - m)
            pr = pr / jnp.sum(pr, axis=1, keepdims=True)
            o_ref[sl, :] = _dotg(pr.astype(BF16), vv).astype(BF16)

    return pl.pallas_call(
        body,
        grid=(heads,),
        in_specs=[
            pl.BlockSpec((t, ep), lambda h: (0, h)),
            pl.BlockSpec((t, ep), lambda h: (0, heads + h)),
            pl.BlockSpec((t, ep), lambda h: (0, 2 * heads + h)),
        ],
        out_specs=pl.BlockSpec((t, ep), lambda h: (0, h)),
        out_shape=jax.ShapeDtypeStruct((t, heads * ep), BF16),
    )(qkv, qkv, qkv)


def _soft_assign(z, clus, bm=512):
    m_, nz = z.shape
    nc = clus.shape[0]

    def body(z_ref, c_ref, o_ref):
        zv = z_ref[...]
        cv = c_ref[...]
        zh, zl2 = _split(zv)
        ch, cl2 = _split(cv)
        cross = (_dotg(zh, ch, True) + _dotg(zh, cl2, True)
                 + _dotg(zl2, ch, True))
        zn = jnp.sum(zv * zv, axis=1, keepdims=True)
        cn = jnp.sum(cv * cv, axis=1)[None, :]
        d2 = zn + cn - 2.0 * cross
        qv = 1.0 / (1.0 + d2)
        o_ref[...] = qv / jnp.sum(qv, axis=1, keepdims=True)

    return pl.pallas_call(
        body,
        grid=(m_ // bm,),
        in_specs=[
            pl.BlockSpec((bm, nz), lambda i: (i, 0)),
            pl.BlockSpec((nc, nz), lambda i: (0, 0)),
        ],
        out_specs=pl.BlockSpec((bm, nc), lambda i: (i, 0)),
        out_shape=jax.ShapeDtypeStruct((m_, nc), F32),
    )(z, clus)


def kernel(x, adj, params):
    p = params
    heads = 8

    x_bf = x.astype(BF16)
    adj_bf = adj.astype(BF16)

    def wbf(name):
        return p[name + "_W"].T.astype(BF16)

    def lin_bn(inp, name, bnname, relu):
        return _mm(inp.astype(BF16), wbf(name), bias=p[name + "_b"],
                   gamma=p[bnname + "_g"], beta=p[bnname + "_b"],
                   epi="relu" if relu else "none")

    e1 = _mm(x_bf, wbf("enc1"), bias=p["enc1_b"],
             gamma=p["bn1_g"], beta=p["bn1_b"], epi="relu")
    e2 = lin_bn(e1, "enc2", "bn2", True)
    e3 = lin_bn(e2, "enc3", "bn3", True)
    z = lin_bn(e3, "zl", "bn4", False)
    d1 = lin_bn(z, "dec1", "bn7", True)
    d2 = lin_bn(d1, "dec2", "bn8", True)
    d3 = lin_bn(d2, "dec3", "bn9", True)
    d3_bf = d3.astype(BF16)
    xbar = _mm(d3_bf, wbf("xbar"), bias=p["xbar_b"])

    _mean = _mm(d3_bf, wbf("dm"), bias=p["dm_b"], epi="expclip")
    _disp = _mm(d3_bf, wbf("dd"), bias=p["dd_b"], epi="softplusclip")
    _pi = _mm(d3_bf, wbf("dp"), bias=p["dp_b"], epi="sigmoid")

    def attn(inp, nm, e, ep, addv):
        def prep(wn):
            wt = p[nm + wn].T.reshape(e, heads, e)
            return jnp.pad(wt, ((0, 0), (0, 0), (0, ep - e))).reshape(
                e, heads * ep)

        wqkv = jnp.concatenate(
            [prep("_q_W"), prep("_k_W"), prep("_v_W")], axis=1).astype(BF16)
        nb = 3 * heads * ep // 1024
        qkv = _mm(inp, wqkv, add=addv, bn=1024, out_dtype=BF16,
                  qk_scale=(float(e) ** 0.25, 2 * nb // 3))
        o = _attention(qkv, heads, ep, inp.shape[0])
        wu = jnp.pad(p[nm + "_u_W"].T.reshape(heads, e, e),
                     ((0, 0), (0, ep - e), (0, 0))).reshape(
                         heads * ep, e).astype(BF16)
        return _mm(o, wu, bias=p[nm + "_u_b"])

    hg1 = _gnn_fused(adj_bf, x_bf, p["g1"].astype(BF16))
    ha3 = attn(hg1, "a3", 500, 512, e2)
    hg3 = _gnn_fused(adj_bf, ha3.astype(BF16), p["g3"].astype(BF16))
    ha5 = attn(hg3, "a5", 64, 128, z)
    h1, predict = _gnn_fused(adj_bf, ha5.astype(BF16), p["g5"].astype(BF16),
                             epi="none", with_softmax=True)

    hd = _gnn_fused(adj_bf, h1, p["g6"].astype(BF16))
    hd = _gnn_fused(adj_bf, (hd + d1).astype(BF16), p["g7"].astype(BF16))
    h = _gnn_fused(adj_bf, (hd + d3).astype(BF16), p["g9"].astype(BF16))

    h_bf = h.astype(BF16)
    a_pred = _mm(h_bf, h_bf, trans_b=True, epi="sigmoid", bn=1024)

    q = _soft_assign(z, p["cluster"])

    return (xbar, q, predict, z, h, a_pred, h1, _mean, _disp, _pi)


# speedup vs baseline: 1.0222x; 1.0222x over previous
import jax
import jax.numpy as jnp
from jax.experimental import pallas as pl

F32 = jnp.float32
BF16 = jnp.bfloat16
BN_EPS = 1e-5


def _split(v):
    hi = v.astype(BF16)
    lo = (v - hi.astype(F32)).astype(BF16)
    return hi, lo


def _dotg(a, b, tb=False):
    dims = (((1,), (1 if tb else 0,)), ((), ()))
    return jax.lax.dot_general(a, b, dims, preferred_element_type=F32)


def _bf(v):
    return v if v.dtype == BF16 else v.astype(BF16)


def _mm(a, b, *, trans_b=False, add=None, bias=None, gamma=None, beta=None,
        epi="none", bk=None, bn=None, out_dtype=F32, qk_scale=None):
    M, K = a.shape
    N = b.shape[0] if trans_b else b.shape[1]
    bk = bk or K
    bn = bn or N
    gk, gn = K // bk, N // bn
    assert out_dtype == F32 or gk == 1

    a_spec = pl.BlockSpec((M, bk), lambda n, k: (0, k))
    if trans_b:
        b_spec = pl.BlockSpec((bn, bk), lambda n, k: (n, k))
    else:
        b_spec = pl.BlockSpec((bk, bn), lambda n, k: (k, n))
    vec_spec = pl.BlockSpec((1, bn), lambda n, k: (0, n))

    in_specs = [a_spec]
    args = [a]
    if add is not None:
        in_specs.append(a_spec)
        args.append(add)
    in_specs.append(b_spec)
    args.append(b)
    for vv in (bias, gamma, beta):
        if vv is not None:
            in_specs.append(vec_spec)
            args.append(vv.reshape(1, N))

    has_add = add is not None
    has_bias = bias is not None
    has_bn = gamma is not None

    def body(*refs):
        refs = list(refs)
        a_ref = refs.pop(0)
        add_ref = refs.pop(0) if has_add else None
        b_ref = refs.pop(0)
        bias_ref = refs.pop(0) if has_bias else None
        g_ref = refs.pop(0) if has_bn else None
        be_ref = refs.pop(0) if has_bn else None
        o_ref = refs.pop(0)

        def epilogue(r):
            if has_bias:
                r = r + bias_ref[...]
            if has_bn:
                mcol = jnp.mean(r, axis=0, keepdims=True)
                vcol = jnp.mean((r - mcol) ** 2, axis=0, keepdims=True)
                r = (r - mcol) / jnp.sqrt(vcol + BN_EPS) * g_ref[...] \
                    + be_ref[...]
            if qk_scale is not None:
                s, nq = qk_scale
                r = jnp.where(pl.program_id(0) < nq, r / s, r)
            if epi == "relu":
                r = jnp.maximum(r, 0.0)
            elif epi == "sigmoid":
                r = jax.nn.sigmoid(r)
            elif epi == "expclip":
                r = jnp.clip(jnp.exp(r), 1e-5, 1e6)
            elif epi == "softplusclip":
                r = jnp.clip(jax.nn.softplus(r), 1e-4, 1e4)
            return r

        av = a_ref[...]
        if has_add:
            av = av + add_ref[...]
        contrib = _dotg(_bf(av), _bf(b_ref[...]), trans_b)

        if gk == 1:
            o_ref[...] = epilogue(contrib).astype(out_dtype)
        else:
            k = pl.program_id(1)

            @pl.when(k == 0)
            def _zero():
                o_ref[...] = jnp.zeros_like(o_ref)

            o_ref[...] += contrib

            @pl.when(k == gk - 1)
            def _epi():
                o_ref[...] = epilogue(o_ref[...])

    return pl.pallas_call(
        body,
        grid=(gn, gk),
        in_specs=in_specs,
        out_specs=pl.BlockSpec((M, bn), lambda n, k: (0, n)),
        out_shape=jax.ShapeDtypeStruct((M, N), out_dtype),
    )(*args)


def _gnn_fused(adj_bf, inp_bf, w_bf, epi="relu", bk=512, with_softmax=False):
    M = adj_bf.shape[0]
    kin = inp_bf.shape[1]
    N = w_bf.shape[1]
    gk = M // bk

    in_specs = [pl.BlockSpec((M, bk), lambda k: (0, k)),
                pl.BlockSpec((bk, kin), lambda k: (k, 0)),
                pl.BlockSpec((kin, N), lambda k: (0, 0))]
    args = [adj_bf, inp_bf, w_bf]

    if with_softmax:
        out_specs = [pl.BlockSpec((M, N), lambda k: (0, 0))] * 2
        out_shape = [jax.ShapeDtypeStruct((M, N), F32)] * 2
    else:
        out_specs = pl.BlockSpec((M, N), lambda k: (0, 0))
        out_shape = jax.ShapeDtypeStruct((M, N), F32)

    def body(adj_ref, inp_ref, w_ref, o_ref, *rest):
        k = pl.program_id(0)

        @pl.when(k == 0)
        def _zero():
            o_ref[...] = jnp.zeros_like(o_ref)

        t_blk = _dotg(_bf(inp_ref[...]), w_ref[...])
        o_ref[...] += _dotg(adj_ref[...], t_blk.astype(BF16))

        @pl.when(k == gk - 1)
        def _epi():
            r = o_ref[...]
            if epi == "relu":
                r = jnp.maximum(r, 0.0)
            o_ref[...] = r
            if with_softmax:
                mx = jnp.max(r, axis=1, keepdims=True)
                pr = jnp.exp(r - mx)
                rest[0][...] = pr / jnp.sum(pr, axis=1, keepdims=True)

    return pl.pallas_call(
        body,
        grid=(gk,),
        in_specs=in_specs,
        out_specs=out_specs,
        out_shape=out_shape,
    )(*args)


def _attention(qkv, heads, ep, t, bq=512):
    def body(q_ref, k_ref, v_ref, o_ref):
        kv = k_ref[...]
        vv = v_ref[...]
        for i in range(t // bq):
            sl = pl.ds(i * bq, bq)
            s = _dotg(q_ref[sl, :], kv, True)
            m = jnp.max(s, axis=1, keepdims=True)
            pr = jnp.exp(s - m)
            pr = pr / jnp.sum(pr, axis=1, keepdims=True)
            o_ref[sl, :] = _dotg(pr.astype(BF16), vv).astype(BF16)

    return pl.pallas_call(
        body,
        grid=(heads,),
        in_specs=[
            pl.BlockSpec((t, ep), lambda h: (0, h)),
            pl.BlockSpec((t, ep), lambda h: (0, heads + h)),
            pl.BlockSpec((t, ep), lambda h: (0, 2 * heads + h)),
        ],
        out_specs=pl.BlockSpec((t, ep), lambda h: (0, h)),
        out_shape=jax.ShapeDtypeStruct((t, heads * ep), BF16),
    )(qkv, qkv, qkv)


def _soft_assign(z, clus, bm=512):
    m_, nz = z.shape
    nc = clus.shape[0]

    def body(z_ref, c_ref, o_ref):
        zv = z_ref[...]
        cv = c_ref[...]
        zh, zl2 = _split(zv)
        ch, cl2 = _split(cv)
        cross = (_dotg(zh, ch, True) + _dotg(zh, cl2, True)
                 + _dotg(zl2, ch, True))
        zn = jnp.sum(zv * zv, axis=1, keepdims=True)
        cn = jnp.sum(cv * cv, axis=1)[None, :]
        d2 = zn + cn - 2.0 * cross
        qv = 1.0 / (1.0 + d2)
        o_ref[...] = qv / jnp.sum(qv, axis=1, keepdims=True)

    return pl.pallas_call(
        body,
        grid=(m_ // bm,),
        in_specs=[
            pl.BlockSpec((bm, nz), lambda i: (i, 0)),
            pl.BlockSpec((nc, nz), lambda i: (0, 0)),
        ],
        out_specs=pl.BlockSpec((bm, nc), lambda i: (i, 0)),
        out_shape=jax.ShapeDtypeStruct((m_, nc), F32),
    )(z, clus)


def kernel(x, adj, params):
    p = params
    heads = 8

    x_bf = x.astype(BF16)
    adj_bf = adj.astype(BF16)

    def wbf(name):
        return p[name + "_W"].T.astype(BF16)

    def lin_bn(inp, name, bnname, relu):
        return _mm(inp.astype(BF16), wbf(name), bias=p[name + "_b"],
                   gamma=p[bnname + "_g"], beta=p[bnname + "_b"],
                   epi="relu" if relu else "none")

    e1 = _mm(x_bf, wbf("enc1"), bias=p["enc1_b"],
             gamma=p["bn1_g"], beta=p["bn1_b"], epi="relu")
    e2 = lin_bn(e1, "enc2", "bn2", True)
    e3 = lin_bn(e2, "enc3", "bn3", True)
    z = lin_bn(e3, "zl", "bn4", False)
    d1 = lin_bn(z, "dec1", "bn7", True)
    d2 = lin_bn(d1, "dec2", "bn8", True)
    d3 = lin_bn(d2, "dec3", "bn9", True)
    d3_bf = d3.astype(BF16)
    xbar = _mm(d3_bf, wbf("xbar"), bias=p["xbar_b"])

    _mean = _mm(d3_bf, wbf("dm"), bias=p["dm_b"], epi="expclip")
    _disp = _mm(d3_bf, wbf("dd"), bias=p["dd_b"], epi="softplusclip")
    _pi = _mm(d3_bf, wbf("dp"), bias=p["dp_b"], epi="sigmoid")

    def attn(inp, nm, e, ep, addv):
        def prep(wn):
            wt = p[nm + wn].T.reshape(e, heads, e)
            return jnp.pad(wt, ((0, 0), (0, 0), (0, ep - e))).reshape(
                e, heads * ep)

        wqkv = jnp.concatenate(
            [prep("_q_W"), prep("_k_W"), prep("_v_W")], axis=1).astype(BF16)
        nb = 3 * heads * ep // 1024
        qkv = _mm(inp, wqkv, add=addv, bn=1024, out_dtype=BF16,
                  qk_scale=(float(e) ** 0.25, 2 * nb // 3))
        o = _attention(qkv, heads, ep, inp.shape[0])
        wu = jnp.pad(p[nm + "_u_W"].T.reshape(heads, e, e),
                     ((0, 0), (0, ep - e), (0, 0))).reshape(
                         heads * ep, e).astype(BF16)
        return _mm(o, wu, bias=p[nm + "_u_b"])

    hg1 = _gnn_fused(adj_bf, x_bf, p["g1"].astype(BF16))
    ha3 = attn(hg1, "a3", 500, 512, e2)
    hg3 = _gnn_fused(adj_bf, ha3.astype(BF16), p["g3"].astype(BF16))
    ha5 = attn(hg3, "a5", 64, 128, z)
    h1, predict = _gnn_fused(adj_bf, ha5.astype(BF16), p["g5"].astype(BF16),
                             epi="none", with_softmax=True)

    hd = _gnn_fused(adj_bf, h1, p["g6"].astype(BF16))
    hd = _gnn_fused(adj_bf, (hd + d1).astype(BF16), p["g7"].astype(BF16))
    h = _gnn_fused(adj_bf, (hd + d3).astype(BF16), p["g9"].astype(BF16))

    h_bf = h.astype(BF16)
    a_pred = _mm(h_bf, h_bf, trans_b=True, epi="sigmoid", bn=1024)

    q = _soft_assign(z, p["cluster"])

    return (xbar, q, predict, z, h, a_pred, h1, _mean, _disp, _pi)


# speedup vs baseline: 1.0240x; 1.0017x over previous
import jax
import jax.numpy as jnp
from jax.experimental import pallas as pl

F32 = jnp.float32
BF16 = jnp.bfloat16
BN_EPS = 1e-5


def _split(v):
    hi = v.astype(BF16)
    lo = (v - hi.astype(F32)).astype(BF16)
    return hi, lo


def _dotg(a, b, tb=False):
    dims = (((1,), (1 if tb else 0,)), ((), ()))
    return jax.lax.dot_general(a, b, dims, preferred_element_type=F32)


def _bf(v):
    return v if v.dtype == BF16 else v.astype(BF16)


def _mm(a, b, *, trans_b=False, add=None, bias=None, gamma=None, beta=None,
        epi="none", bk=None, bn=None, out_dtype=F32, qk_scale=None):
    M, K = a.shape
    N = b.shape[0] if trans_b else b.shape[1]
    bk = bk or K
    bn = bn or N
    gk, gn = K // bk, N // bn
    assert out_dtype == F32 or gk == 1

    a_spec = pl.BlockSpec((M, bk), lambda n, k: (0, k))
    if trans_b:
        b_spec = pl.BlockSpec((bn, bk), lambda n, k: (n, k))
    else:
        b_spec = pl.BlockSpec((bk, bn), lambda n, k: (k, n))
    vec_spec = pl.BlockSpec((1, bn), lambda n, k: (0, n))

    in_specs = [a_spec]
    args = [a]
    if add is not None:
        in_specs.append(a_spec)
        args.append(add)
    in_specs.append(b_spec)
    args.append(b)
    for vv in (bias, gamma, beta):
        if vv is not None:
            in_specs.append(vec_spec)
            args.append(vv.reshape(1, N))

    has_add = add is not None
    has_bias = bias is not None
    has_bn = gamma is not None

    def body(*refs):
        refs = list(refs)
        a_ref = refs.pop(0)
        add_ref = refs.pop(0) if has_add else None
        b_ref = refs.pop(0)
        bias_ref = refs.pop(0) if has_bias else None
        g_ref = refs.pop(0) if has_bn else None
        be_ref = refs.pop(0) if has_bn else None
        o_ref = refs.pop(0)

        def epilogue(r):
            if has_bias:
                r = r + bias_ref[...]
            if has_bn:
                mcol = jnp.mean(r, axis=0, keepdims=True)
                vcol = jnp.mean((r - mcol) ** 2, axis=0, keepdims=True)
                r = (r - mcol) / jnp.sqrt(vcol + BN_EPS) * g_ref[...] \
                    + be_ref[...]
            if qk_scale is not None:
                s, nq = qk_scale
                r = jnp.where(pl.program_id(0) < nq, r / s, r)
            if epi == "relu":
                r = jnp.maximum(r, 0.0)
            elif epi == "sigmoid":
                r = jax.nn.sigmoid(r)
            elif epi == "expclip":
                r = jnp.clip(jnp.exp(r), 1e-5, 1e6)
            elif epi == "softplusclip":
                r = jnp.clip(jax.nn.softplus(r), 1e-4, 1e4)
            return r

        av = a_ref[...]
        if has_add:
            av = av + add_ref[...]
        contrib = _dotg(_bf(av), _bf(b_ref[...]), trans_b)

        if gk == 1:
            o_ref[...] = epilogue(contrib).astype(out_dtype)
        else:
            k = pl.program_id(1)

            @pl.when(k == 0)
            def _zero():
                o_ref[...] = jnp.zeros_like(o_ref)

            o_ref[...] += contrib

            @pl.when(k == gk - 1)
            def _epi():
                o_ref[...] = epilogue(o_ref[...])

    return pl.pallas_call(
        body,
        grid=(gn, gk),
        in_specs=in_specs,
        out_specs=pl.BlockSpec((M, bn), lambda n, k: (0, n)),
        out_shape=jax.ShapeDtypeStruct((M, N), out_dtype),
    )(*args)


def _gnn_fused(adj_bf, inp_bf, w_bf, epi="relu", bk=1024, with_softmax=False):
    M = adj_bf.shape[0]
    kin = inp_bf.shape[1]
    N = w_bf.shape[1]
    gk = M // bk

    in_specs = [pl.BlockSpec((M, bk), lambda k: (0, k)),
                pl.BlockSpec((bk, kin), lambda k: (k, 0)),
                pl.BlockSpec((kin, N), lambda k: (0, 0))]
    args = [adj_bf, inp_bf, w_bf]

    if with_softmax:
        out_specs = [pl.BlockSpec((M, N), lambda k: (0, 0))] * 2
        out_shape = [jax.ShapeDtypeStruct((M, N), F32)] * 2
    else:
        out_specs = pl.BlockSpec((M, N), lambda k: (0, 0))
        out_shape = jax.ShapeDtypeStruct((M, N), F32)

    def body(adj_ref, inp_ref, w_ref, o_ref, *rest):
        k = pl.program_id(0)

        @pl.when(k == 0)
        def _zero():
            o_ref[...] = jnp.zeros_like(o_ref)

        t_blk = _dotg(_bf(inp_ref[...]), w_ref[...])
        o_ref[...] += _dotg(adj_ref[...], t_blk.astype(BF16))

        @pl.when(k == gk - 1)
        def _epi():
            r = o_ref[...]
            if epi == "relu":
                r = jnp.maximum(r, 0.0)
            o_ref[...] = r
            if with_softmax:
                mx = jnp.max(r, axis=1, keepdims=True)
                pr = jnp.exp(r - mx)
                rest[0][...] = pr / jnp.sum(pr, axis=1, keepdims=True)

    return pl.pallas_call(
        body,
        grid=(gk,),
        in_specs=in_specs,
        out_specs=out_specs,
        out_shape=out_shape,
    )(*args)


def _attention(qkv, heads, ep, t, bq=512):
    def body(q_ref, k_ref, v_ref, o_ref):
        kv = k_ref[...]
        vv = v_ref[...]
        for i in range(t // bq):
            sl = pl.ds(i * bq, bq)
            s = _dotg(q_ref[sl, :], kv, True)
            m = jnp.max(s, axis=1, keepdims=True)
            pr = jnp.exp(s - m)
            pr = pr / jnp.sum(pr, axis=1, keepdims=True)
            o_ref[sl, :] = _dotg(pr.astype(BF16), vv).astype(BF16)

    return pl.pallas_call(
        body,
        grid=(heads,),
        in_specs=[
            pl.BlockSpec((t, ep), lambda h: (0, h)),
            pl.BlockSpec((t, ep), lambda h: (0, heads + h)),
            pl.BlockSpec((t, ep), lambda h: (0, 2 * heads + h)),
        ],
        out_specs=pl.BlockSpec((t, ep), lambda h: (0, h)),
        out_shape=jax.ShapeDtypeStruct((t, heads * ep), BF16),
    )(qkv, qkv, qkv)


def _soft_assign(z, clus, bm=512):
    m_, nz = z.shape
    nc = clus.shape[0]

    def body(z_ref, c_ref, o_ref):
        zv = z_ref[...]
        cv = c_ref[...]
        zh, zl2 = _split(zv)
        ch, cl2 = _split(cv)
        cross = (_dotg(zh, ch, True) + _dotg(zh, cl2, True)
                 + _dotg(zl2, ch, True))
        zn = jnp.sum(zv * zv, axis=1, keepdims=True)
        cn = jnp.sum(cv * cv, axis=1)[None, :]
        d2 = zn + cn - 2.0 * cross
        qv = 1.0 / (1.0 + d2)
        o_ref[...] = qv / jnp.sum(qv, axis=1, keepdims=True)

    return pl.pallas_call(
        body,
        grid=(m_ // bm,),
        in_specs=[
            pl.BlockSpec((bm, nz), lambda i: (i, 0)),
            pl.BlockSpec((nc, nz), lambda i: (0, 0)),
        ],
        out_specs=pl.BlockSpec((bm, nc), lambda i: (i, 0)),
        out_shape=jax.ShapeDtypeStruct((m_, nc), F32),
    )(z, clus)


def kernel(x, adj, params):
    p = params
    heads = 8

    x_bf = x.astype(BF16)
    adj_bf = adj.astype(BF16)

    def wbf(name):
        return p[name + "_W"].T.astype(BF16)

    def lin_bn(inp, name, bnname, relu):
        return _mm(inp.astype(BF16), wbf(name), bias=p[name + "_b"],
                   gamma=p[bnname + "_g"], beta=p[bnname + "_b"],
                   epi="relu" if relu else "none")

    e1 = _mm(x_bf, wbf("enc1"), bias=p["enc1_b"],
             gamma=p["bn1_g"], beta=p["bn1_b"], epi="relu")
    e2 = lin_bn(e1, "enc2", "bn2", True)
    e3 = lin_bn(e2, "enc3", "bn3", True)
    z = lin_bn(e3, "zl", "bn4", False)
    d1 = lin_bn(z, "dec1", "bn7", True)
    d2 = lin_bn(d1, "dec2", "bn8", True)
    d3 = lin_bn(d2, "dec3", "bn9", True)
    d3_bf = d3.astype(BF16)
    xbar = _mm(d3_bf, wbf("xbar"), bias=p["xbar_b"])

    _mean = _mm(d3_bf, wbf("dm"), bias=p["dm_b"], epi="expclip")
    _disp = _mm(d3_bf, wbf("dd"), bias=p["dd_b"], epi="softplusclip")
    _pi = _mm(d3_bf, wbf("dp"), bias=p["dp_b"], epi="sigmoid")

    def attn(inp, nm, e, ep, addv):
        def prep(wn):
            wt = p[nm + wn].T.reshape(e, heads, e)
            return jnp.pad(wt, ((0, 0), (0, 0), (0, ep - e))).reshape(
                e, heads * ep)

        wqkv = jnp.concatenate(
            [prep("_q_W"), prep("_k_W"), prep("_v_W")], axis=1).astype(BF16)
        nb = 3 * heads * ep // 1024
        qkv = _mm(inp, wqkv, add=addv, bn=1024, out_dtype=BF16,
                  qk_scale=(float(e) ** 0.25, 2 * nb // 3))
        o = _attention(qkv, heads, ep, inp.shape[0])
        wu = jnp.pad(p[nm + "_u_W"].T.reshape(heads, e, e),
                     ((0, 0), (0, ep - e), (0, 0))).reshape(
                         heads * ep, e).astype(BF16)
        return _mm(o, wu, bias=p[nm + "_u_b"])

    hg1 = _gnn_fused(adj_bf, x_bf, p["g1"].astype(BF16))
    ha3 = attn(hg1, "a3", 500, 512, e2)
    hg3 = _gnn_fused(adj_bf, ha3.astype(BF16), p["g3"].astype(BF16))
    ha5 = attn(hg3, "a5", 64, 128, z)
    h1, predict = _gnn_fused(adj_bf, ha5.astype(BF16), p["g5"].astype(BF16),
                             epi="none", with_softmax=True)

    hd = _gnn_fused(adj_bf, h1, p["g6"].astype(BF16))
    hd = _gnn_fused(adj_bf, (hd + d1).astype(BF16), p["g7"].astype(BF16))
    h = _gnn_fused(adj_bf, (hd + d3).astype(BF16), p["g9"].astype(BF16))

    h_bf = h.astype(BF16)
    a_pred = _mm(h_bf, h_bf, trans_b=True, epi="sigmoid", bn=1024)

    q = _soft_assign(z, p["cluster"])

    return (xbar, q, predict, z, h, a_pred, h1, _mean, _disp, _pi)
